# baseline (speedup 1.0000x reference)
import functools
import math

import jax
import jax.numpy as jnp
from jax import lax
from jax.experimental import pallas as pl
from jax.experimental.pallas import tpu as pltpu
from jax.experimental.pallas import tpu_sc as plsc

_W = 131072


def _tc_pack(qt_t, lin_t):
    k, r = qt_t.shape
    g = math.ceil(r / _W)
    sw = _W // 8

    def body(q_ref, l_ref, q_out, l_out):
        x = q_ref[...]
        xs = jnp.concatenate(
            [x[:, s * sw:(s + 1) * sw] for s in range(8)], axis=0)
        q_out[...] = jnp.transpose(xs)
        z = l_ref[...]
        l_out[...] = jnp.concatenate(
            [z[:, c * 128:(c + 1) * 128] for c in range(_W // 128)], axis=0)

    return pl.pallas_call(
        body,
        grid=(g,),
        in_specs=[pl.BlockSpec((k, _W), lambda i: (0, i)),
                  pl.BlockSpec((1, _W), lambda i: (0, i))],
        out_specs=[pl.BlockSpec((_W // 8, 128), lambda i: (i, 0)),
                   pl.BlockSpec((_W // 128, 128), lambda i: (i, 0))],
        out_shape=[jax.ShapeDtypeStruct((g * _W // 8, 128), jnp.float32),
                   jax.ShapeDtypeStruct((g * _W // 128, 128), jnp.float32)],
        compiler_params=pltpu.CompilerParams(
            dimension_semantics=("arbitrary",)),
    )(qt_t, lin_t)


def _sc_gather(idx3d, q128, l128, n_fields, n_bblk):
    nw, c_per_w = idx3d.shape[0], idx3d.shape[1]
    info = plsc.get_sparse_core_info()
    nc = info.num_cores
    assert nc * info.num_subcores == nw

    mesh = plsc.VectorSubcoreMesh(core_axis_name="c", subcore_axis_name="s")

    @functools.partial(
        pl.kernel,
        mesh=mesh,
        compiler_params=pltpu.CompilerParams(use_tc_tiling_on_sc=False,
                                             needs_layout_passes=False),
        out_type=[
            jax.ShapeDtypeStruct((n_fields, n_bblk, 16, 128), jnp.float32),
            jax.ShapeDtypeStruct((n_fields, n_bblk, 1, 128), jnp.float32),
        ],
        scratch_types=[
            pltpu.VMEM((c_per_w, 128), jnp.int32),
            pltpu.VMEM((c_per_w, 128), jnp.int32),
            pltpu.VMEM((c_per_w, 128), jnp.int32),
            pltpu.VMEM((4, 128, 16), jnp.float32),
            pltpu.VMEM((4, 128, 16), jnp.float32),
            pltpu.VMEM((16, 128), jnp.float32),
            pltpu.VMEM((1, 128), jnp.float32),
            pltpu.SemaphoreType.DMA((4,)),
            pltpu.SemaphoreType.DMA((4,)),
        ],
    )
    def gather_kernel(idx_hbm, q_hbm, l_hbm, emb_out, lin_out,
                      idx_v, qidx_v, lidx_v, qbuf, lbuf, ebuf, lvbuf,
                      sem_q, sem_l):
        wid = lax.axis_index("s") * nc + lax.axis_index("c")
        base = wid * c_per_w
        pltpu.sync_copy(idx_hbm.at[wid], idx_v)
        iota16 = lax.iota(jnp.int32, 16)

        def precomp(g, carry):
            for j in range(8):
                v = idx_v[g, pl.ds(j * 16, 16)]
                qidx_v[g, pl.ds(j * 16, 16)] = (
                    lax.shift_left(lax.shift_right_logical(v, 17), 17)
                    | lax.shift_left(lax.bitwise_and(v, 16383), 3)
                    | lax.bitwise_and(lax.shift_right_logical(v, 14), 7))
                lidx_v[g, pl.ds(j * 16, 16)] = lax.shift_right_logical(v, 4)
            return carry

        lax.fori_loop(0, c_per_w, precomp, 0)

        def start(g, slot):
            pltpu.async_copy(q_hbm.at[qidx_v.at[g]], qbuf.at[slot],
                             sem_q.at[slot])
            pltpu.async_copy(l_hbm.at[lidx_v.at[g]], lbuf.at[slot],
                             sem_l.at[slot])

        for pg in range(3):
            start(pg, pg)

        def step(g, carry):
            slot = lax.bitwise_and(g, 3)

            @pl.when(g + 3 < c_per_w)
            def _():
                start(g + 3, lax.bitwise_and(g + 3, 3))

            pltpu.make_async_copy(q_hbm.at[qidx_v.at[g]], qbuf.at[slot],
                                  sem_q.at[slot]).wait()
            pltpu.make_async_copy(l_hbm.at[lidx_v.at[g]], lbuf.at[slot],
                                  sem_l.at[slot]).wait()
            for j in range(8):
                v = idx_v[g, pl.ds(j * 16, 16)]
                rows = iota16 + j * 16
                for k in range(16):
                    ebuf[k, pl.ds(j * 16, 16)] = plsc.load_gather(
                        qbuf.at[slot], [rows, iota16 * 0 + k])
                lvbuf[0, pl.ds(j * 16, 16)] = plsc.load_gather(
                    lbuf.at[slot], [rows, lax.bitwise_and(v, 15)])
            c = base + g
            f = lax.div(c, n_bblk)
            bb = lax.rem(c, n_bblk)
            pltpu.sync_copy(ebuf, emb_out.at[f, bb])
            pltpu.sync_copy(lvbuf, lin_out.at[f, bb])
            return carry

        lax.fori_loop(0, c_per_w, step, 0)

    return gather_kernel(idx3d, q128, l128)


def _tc_head(emb4, lin3, s_t, w1t, b1c, w2t, b2c, w3c, cbias):
    f, n_bblk = emb4.shape[0], emb4.shape[1]
    d_in = f * 16
    h1 = w1t.shape[0]
    h2 = w2t.shape[0]

    nb = 4

    def body(emb_ref, lin_ref, s_ref, w1_ref, b1_ref, w2_ref, b2_ref,
             w3_ref, cb_ref, out_ref):
        x = jnp.concatenate(
            [emb_ref[:, q].reshape(d_in, 128) for q in range(nb)],
            axis=1)
        ksum = jnp.dot(s_ref[...], x, preferred_element_type=jnp.float32)
        sq_sum = jnp.sum(ksum * ksum, axis=0, keepdims=True)
        sum_sq = jnp.sum(x * x, axis=0, keepdims=True)
        quad = 0.5 * (sq_sum - sum_sq)
        lin = jnp.concatenate(
            [jnp.sum(lin_ref[:, q, 0, :], axis=0, keepdims=True)
             for q in range(nb)], axis=1)
        h = jnp.dot(w1_ref[...], x, preferred_element_type=jnp.float32)
        h = jnp.maximum(h + b1_ref[...], 0.0)
        h = jnp.dot(w2_ref[...], h, preferred_element_type=jnp.float32)
        h = jnp.maximum(h + b2_ref[...], 0.0)
        ymlp = jnp.sum(h * w3_ref[...], axis=0, keepdims=True)
        z = cb_ref[...] + lin + quad + ymlp
        out_ref[...] = (1.0 / (1.0 + jnp.exp(-z))).reshape(1, 1, nb * 128)

    return pl.pallas_call(
        body,
        grid=(n_bblk // 4,),
        in_specs=[
            pl.BlockSpec((f, 4, 16, 128), lambda i: (0, i, 0, 0)),
            pl.BlockSpec((f, 4, 1, 128), lambda i: (0, i, 0, 0)),
            pl.BlockSpec((16, d_in), lambda i: (0, 0)),
            pl.BlockSpec((h1, d_in), lambda i: (0, 0)),
            pl.BlockSpec((h1, 1), lambda i: (0, 0)),
            pl.BlockSpec((h2, h1), lambda i: (0, 0)),
            pl.BlockSpec((h2, 1), lambda i: (0, 0)),
            pl.BlockSpec((h2, 1), lambda i: (0, 0)),
            pl.BlockSpec((1, 1), lambda i: (0, 0)),
        ],
        out_specs=pl.BlockSpec((1, 1, 512), lambda i: (i, 0, 0)),
        out_shape=jax.ShapeDtypeStruct((n_bblk // 4, 1, 512), jnp.float32),
    )(emb4, lin3, s_t, w1t, b1c, w2t, b2c, w3c, cbias)


def kernel(input, quad_table, lin_table, global_bias, W1, b1, W2, b2, W3, b3):
    b, f = input.shape
    r, k = quad_table.shape
    vocab = r // f
    nw = 32
    n_bblk = b // 128
    offsets = jnp.arange(f, dtype=input.dtype) * vocab
    idx_fm = input.T + offsets[:, None]
    idx3d = idx_fm.reshape(nw, -1, 128)
    q128, l128 = _tc_pack(quad_table.T, lin_table.T)
    emb4, lin3 = _sc_gather(idx3d, q128.reshape(-1, 16),
                            l128.reshape(-1, 16), f, n_bblk)
    s_t = jnp.tile(jnp.eye(k, dtype=jnp.float32), (1, f))
    cbias = (global_bias[0] + b3[0]).reshape(1, 1)
    out = _tc_head(emb4, lin3, s_t, W1.T, b1.reshape(-1, 1), W2.T,
                   b2.reshape(-1, 1), W3, cbias)
    return out.reshape(b)


# speedup vs baseline: 1.0283x; 1.0283x over previous
import functools
import math

import jax
import jax.numpy as jnp
from jax import lax
from jax.experimental import pallas as pl
from jax.experimental.pallas import tpu as pltpu
from jax.experimental.pallas import tpu_sc as plsc

_W = 131072


def _tc_pack(qt_t, lin_t):
    k, r = qt_t.shape
    g = math.ceil(r / _W)
    sw = _W // 8

    def body(q_ref, l_ref, q_out, l_out):
        x = q_ref[...]
        xs = jnp.concatenate(
            [x[:, s * sw:(s + 1) * sw] for s in range(8)], axis=0)
        q_out[...] = jnp.transpose(xs)
        z = l_ref[...]
        l_out[...] = jnp.concatenate(
            [z[:, c * 128:(c + 1) * 128] for c in range(_W // 128)], axis=0)

    return pl.pallas_call(
        body,
        grid=(g,),
        in_specs=[pl.BlockSpec((k, _W), lambda i: (0, i)),
                  pl.BlockSpec((1, _W), lambda i: (0, i))],
        out_specs=[pl.BlockSpec((_W // 8, 128), lambda i: (i, 0)),
                   pl.BlockSpec((_W // 128, 128), lambda i: (i, 0))],
        out_shape=[jax.ShapeDtypeStruct((g * _W // 8, 128), jnp.float32),
                   jax.ShapeDtypeStruct((g * _W // 128, 128), jnp.float32)],
        compiler_params=pltpu.CompilerParams(
            dimension_semantics=("arbitrary",)),
    )(qt_t, lin_t)


def _sc_gather(idx3d, q128, l128, n_fields, n_bblk):
    nw, c_per_w = idx3d.shape[0], idx3d.shape[1]
    info = plsc.get_sparse_core_info()
    nc = info.num_cores
    assert nc * info.num_subcores == nw

    mesh = plsc.VectorSubcoreMesh(core_axis_name="c", subcore_axis_name="s")

    @functools.partial(
        pl.kernel,
        mesh=mesh,
        compiler_params=pltpu.CompilerParams(use_tc_tiling_on_sc=False,
                                             needs_layout_passes=False),
        out_type=[
            jax.ShapeDtypeStruct((n_fields, n_bblk, 16, 128), jnp.float32),
            jax.ShapeDtypeStruct((n_fields, n_bblk, 1, 128), jnp.float32),
        ],
        scratch_types=[
            pltpu.VMEM((c_per_w, 128), jnp.int32),
            pltpu.VMEM((c_per_w, 128), jnp.int32),
            pltpu.VMEM((c_per_w, 128), jnp.int32),
            pltpu.VMEM((4, 128, 16), jnp.float32),
            pltpu.VMEM((4, 128, 16), jnp.float32),
            pltpu.VMEM((4, 16, 128), jnp.float32),
            pltpu.VMEM((4, 1, 128), jnp.float32),
            pltpu.SemaphoreType.DMA((4,)),
            pltpu.SemaphoreType.DMA((4,)),
            pltpu.SemaphoreType.DMA((4,)),
            pltpu.SemaphoreType.DMA((4,)),
        ],
    )
    def gather_kernel(idx_hbm, q_hbm, l_hbm, emb_out, lin_out,
                      idx_v, qidx_v, lidx_v, qbuf, lbuf, ebuf, lvbuf,
                      sem_q, sem_l, sem_eo, sem_lo):
        wid = lax.axis_index("s") * nc + lax.axis_index("c")
        base = wid * c_per_w
        pltpu.sync_copy(idx_hbm.at[wid], idx_v)
        iota16 = lax.iota(jnp.int32, 16)

        def precomp(g, carry):
            for j in range(8):
                v = idx_v[g, pl.ds(j * 16, 16)]
                qidx_v[g, pl.ds(j * 16, 16)] = (
                    lax.shift_left(lax.shift_right_logical(v, 17), 17)
                    | lax.shift_left(lax.bitwise_and(v, 16383), 3)
                    | lax.bitwise_and(lax.shift_right_logical(v, 14), 7))
                lidx_v[g, pl.ds(j * 16, 16)] = lax.shift_right_logical(v, 4)
            return carry

        lax.fori_loop(0, c_per_w, precomp, 0)

        def start(g, slot):
            pltpu.async_copy(q_hbm.at[qidx_v.at[g]], qbuf.at[slot],
                             sem_q.at[slot])
            pltpu.async_copy(l_hbm.at[lidx_v.at[g]], lbuf.at[slot],
                             sem_l.at[slot])

        for pg in range(3):
            start(pg, pg)

        def step(g, carry):
            slot = lax.bitwise_and(g, 3)

            @pl.when(g + 3 < c_per_w)
            def _():
                start(g + 3, lax.bitwise_and(g + 3, 3))

            pltpu.make_async_copy(q_hbm.at[qidx_v.at[g]], qbuf.at[slot],
                                  sem_q.at[slot]).wait()
            pltpu.make_async_copy(l_hbm.at[lidx_v.at[g]], lbuf.at[slot],
                                  sem_l.at[slot]).wait()

            @pl.when(g >= 4)
            def _():
                cp = base + g - 4
                fp = lax.div(cp, n_bblk)
                bp = lax.rem(cp, n_bblk)
                pltpu.make_async_copy(ebuf.at[slot], emb_out.at[fp, bp],
                                      sem_eo.at[slot]).wait()
                pltpu.make_async_copy(lvbuf.at[slot], lin_out.at[fp, bp],
                                      sem_lo.at[slot]).wait()

            for j in range(8):
                v = idx_v[g, pl.ds(j * 16, 16)]
                rows = iota16 + j * 16
                for k in range(16):
                    ebuf[slot, k, pl.ds(j * 16, 16)] = plsc.load_gather(
                        qbuf.at[slot], [rows, iota16 * 0 + k])
                lvbuf[slot, 0, pl.ds(j * 16, 16)] = plsc.load_gather(
                    lbuf.at[slot], [rows, lax.bitwise_and(v, 15)])
            c = base + g
            f = lax.div(c, n_bblk)
            bb = lax.rem(c, n_bblk)
            pltpu.async_copy(ebuf.at[slot], emb_out.at[f, bb],
                             sem_eo.at[slot])
            pltpu.async_copy(lvbuf.at[slot], lin_out.at[f, bb],
                             sem_lo.at[slot])
            return carry

        lax.fori_loop(0, c_per_w, step, 0)
        for s in range(4):
            gd = c_per_w - 4 + s
            cd = base + gd
            fd = lax.div(cd, n_bblk)
            bd = lax.rem(cd, n_bblk)
            pltpu.make_async_copy(ebuf.at[gd & 3], emb_out.at[fd, bd],
                                  sem_eo.at[gd & 3]).wait()
            pltpu.make_async_copy(lvbuf.at[gd & 3], lin_out.at[fd, bd],
                                  sem_lo.at[gd & 3]).wait()

    return gather_kernel(idx3d, q128, l128)


def _tc_head(emb4, lin3, s_t, w1t, b1c, w2t, b2c, w3c, cbias):
    f, n_bblk = emb4.shape[0], emb4.shape[1]
    d_in = f * 16
    h1 = w1t.shape[0]
    h2 = w2t.shape[0]

    nb = 4

    def body(emb_ref, lin_ref, s_ref, w1_ref, b1_ref, w2_ref, b2_ref,
             w3_ref, cb_ref, out_ref):
        x = jnp.concatenate(
            [emb_ref[:, q].reshape(d_in, 128) for q in range(nb)],
            axis=1)
        ksum = jnp.dot(s_ref[...], x, preferred_element_type=jnp.float32)
        sq_sum = jnp.sum(ksum * ksum, axis=0, keepdims=True)
        sum_sq = jnp.sum(x * x, axis=0, keepdims=True)
        quad = 0.5 * (sq_sum - sum_sq)
        lin = jnp.concatenate(
            [jnp.sum(lin_ref[:, q, 0, :], axis=0, keepdims=True)
             for q in range(nb)], axis=1)
        h = jnp.dot(w1_ref[...], x, preferred_element_type=jnp.float32)
        h = jnp.maximum(h + b1_ref[...], 0.0)
        h = jnp.dot(w2_ref[...], h, preferred_element_type=jnp.float32)
        h = jnp.maximum(h + b2_ref[...], 0.0)
        ymlp = jnp.sum(h * w3_ref[...], axis=0, keepdims=True)
        z = cb_ref[...] + lin + quad + ymlp
        out_ref[...] = (1.0 / (1.0 + jnp.exp(-z))).reshape(1, 1, nb * 128)

    return pl.pallas_call(
        body,
        grid=(n_bblk // 4,),
        in_specs=[
            pl.BlockSpec((f, 4, 16, 128), lambda i: (0, i, 0, 0)),
            pl.BlockSpec((f, 4, 1, 128), lambda i: (0, i, 0, 0)),
            pl.BlockSpec((16, d_in), lambda i: (0, 0)),
            pl.BlockSpec((h1, d_in), lambda i: (0, 0)),
            pl.BlockSpec((h1, 1), lambda i: (0, 0)),
            pl.BlockSpec((h2, h1), lambda i: (0, 0)),
            pl.BlockSpec((h2, 1), lambda i: (0, 0)),
            pl.BlockSpec((h2, 1), lambda i: (0, 0)),
            pl.BlockSpec((1, 1), lambda i: (0, 0)),
        ],
        out_specs=pl.BlockSpec((1, 1, 512), lambda i: (i, 0, 0)),
        out_shape=jax.ShapeDtypeStruct((n_bblk // 4, 1, 512), jnp.float32),
    )(emb4, lin3, s_t, w1t, b1c, w2t, b2c, w3c, cbias)


def kernel(input, quad_table, lin_table, global_bias, W1, b1, W2, b2, W3, b3):
    b, f = input.shape
    r, k = quad_table.shape
    vocab = r // f
    nw = 32
    n_bblk = b // 128
    offsets = jnp.arange(f, dtype=input.dtype) * vocab
    idx_fm = input.T + offsets[:, None]
    idx3d = idx_fm.reshape(nw, -1, 128)
    q128, l128 = _tc_pack(quad_table.T, lin_table.T)
    emb4, lin3 = _sc_gather(idx3d, q128.reshape(-1, 16),
                            l128.reshape(-1, 16), f, n_bblk)
    s_t = jnp.tile(jnp.eye(k, dtype=jnp.float32), (1, f))
    cbias = (global_bias[0] + b3[0]).reshape(1, 1)
    out = _tc_head(emb4, lin3, s_t, W1.T, b1.reshape(-1, 1), W2.T,
                   b2.reshape(-1, 1), W3, cbias)
    return out.reshape(b)


# speedup vs baseline: 1.0421x; 1.0134x over previous
import functools
import math

import jax
import jax.numpy as jnp
from jax import lax
from jax.experimental import pallas as pl
from jax.experimental.pallas import tpu as pltpu
from jax.experimental.pallas import tpu_sc as plsc

_W = 131072


def _tc_pack(qt_t, lin_t):
    k, r = qt_t.shape
    g = math.ceil(r / _W)
    sw = _W // 8

    def body(q_ref, l_ref, q_out, l_out):
        x = q_ref[...]
        xs = jnp.concatenate(
            [x[:, s * sw:(s + 1) * sw] for s in range(8)], axis=0)
        q_out[...] = jnp.transpose(xs)
        z = l_ref[...]
        l_out[...] = jnp.concatenate(
            [z[:, c * 128:(c + 1) * 128] for c in range(_W // 128)], axis=0)

    return pl.pallas_call(
        body,
        grid=(g,),
        in_specs=[pl.BlockSpec((k, _W), lambda i: (0, i)),
                  pl.BlockSpec((1, _W), lambda i: (0, i))],
        out_specs=[pl.BlockSpec((_W // 8, 128), lambda i: (i, 0)),
                   pl.BlockSpec((_W // 128, 128), lambda i: (i, 0))],
        out_shape=[jax.ShapeDtypeStruct((g * _W // 8, 128), jnp.float32),
                   jax.ShapeDtypeStruct((g * _W // 128, 128), jnp.float32)],
        compiler_params=pltpu.CompilerParams(
            dimension_semantics=("arbitrary",)),
    )(qt_t, lin_t)


def _sc_gather(idx3d, q128, l128, n_fields, n_bblk):
    nw, c_per_w = idx3d.shape[0], idx3d.shape[1]
    info = plsc.get_sparse_core_info()
    nc = info.num_cores
    assert nc * info.num_subcores == nw

    mesh = plsc.VectorSubcoreMesh(core_axis_name="c", subcore_axis_name="s")

    @functools.partial(
        pl.kernel,
        mesh=mesh,
        compiler_params=pltpu.CompilerParams(use_tc_tiling_on_sc=False,
                                             needs_layout_passes=False),
        out_type=[
            jax.ShapeDtypeStruct((n_fields, n_bblk, 16, 128), jnp.float32),
            jax.ShapeDtypeStruct((n_fields, n_bblk, 1, 128), jnp.float32),
        ],
        scratch_types=[
            pltpu.VMEM((c_per_w, 128), jnp.int32),
            pltpu.VMEM((c_per_w, 128), jnp.int32),
            pltpu.VMEM((c_per_w, 128), jnp.int32),
            pltpu.VMEM((4, 128, 16), jnp.float32),
            pltpu.VMEM((4, 128, 16), jnp.float32),
            pltpu.VMEM((4, 16, 128), jnp.float32),
            pltpu.VMEM((4, 1, 128), jnp.float32),
            pltpu.SemaphoreType.DMA((4,)),
            pltpu.SemaphoreType.DMA((4,)),
            pltpu.SemaphoreType.DMA((4,)),
            pltpu.SemaphoreType.DMA((4,)),
        ],
    )
    def gather_kernel(idx_hbm, q_hbm, l_hbm, emb_out, lin_out,
                      idx_v, qidx_v, lidx_v, qbuf, lbuf, ebuf, lvbuf,
                      sem_q, sem_l, sem_eo, sem_lo):
        wid = lax.axis_index("s") * nc + lax.axis_index("c")
        base = wid * c_per_w
        pltpu.sync_copy(idx_hbm.at[wid], idx_v)
        iota16 = lax.iota(jnp.int32, 16)

        def precomp(g, carry):
            for j in range(8):
                v = idx_v[g, pl.ds(j * 16, 16)]
                qidx_v[g, pl.ds(j * 16, 16)] = (
                    lax.shift_left(lax.shift_right_logical(v, 17), 17)
                    | lax.shift_left(lax.bitwise_and(v, 16383), 3)
                    | lax.bitwise_and(lax.shift_right_logical(v, 14), 7))
                lidx_v[g, pl.ds(j * 16, 16)] = lax.shift_right_logical(v, 4)
            return carry

        lax.fori_loop(0, c_per_w, precomp, 0)

        def start(g, slot):
            pltpu.async_copy(q_hbm.at[qidx_v.at[g]], qbuf.at[slot],
                             sem_q.at[slot])
            pltpu.async_copy(l_hbm.at[lidx_v.at[g]], lbuf.at[slot],
                             sem_l.at[slot])

        for pg in range(3):
            start(pg, pg)

        def step(g, carry):
            slot = lax.bitwise_and(g, 3)

            @pl.when(g + 3 < c_per_w)
            def _():
                start(g + 3, lax.bitwise_and(g + 3, 3))

            pltpu.make_async_copy(q_hbm.at[qidx_v.at[g]], qbuf.at[slot],
                                  sem_q.at[slot]).wait()
            pltpu.make_async_copy(l_hbm.at[lidx_v.at[g]], lbuf.at[slot],
                                  sem_l.at[slot]).wait()

            @pl.when(g >= 4)
            def _():
                cp = base + g - 4
                fp = lax.div(cp, n_bblk)
                bp = lax.rem(cp, n_bblk)
                pltpu.make_async_copy(ebuf.at[slot], emb_out.at[fp, bp],
                                      sem_eo.at[slot]).wait()
                pltpu.make_async_copy(lvbuf.at[slot], lin_out.at[fp, bp],
                                      sem_lo.at[slot]).wait()

            for j in range(8):
                v = idx_v[g, pl.ds(j * 16, 16)]
                rows = iota16 + j * 16
                for k in range(16):
                    ebuf[slot, k, pl.ds(j * 16, 16)] = plsc.load_gather(
                        qbuf.at[slot], [rows, iota16 * 0 + k])
                lvbuf[slot, 0, pl.ds(j * 16, 16)] = plsc.load_gather(
                    lbuf.at[slot], [rows, lax.bitwise_and(v, 15)])
            c = base + g
            f = lax.div(c, n_bblk)
            bb = lax.rem(c, n_bblk)
            pltpu.async_copy(ebuf.at[slot], emb_out.at[f, bb],
                             sem_eo.at[slot])
            pltpu.async_copy(lvbuf.at[slot], lin_out.at[f, bb],
                             sem_lo.at[slot])
            return carry

        lax.fori_loop(0, c_per_w, step, 0)
        for s in range(4):
            gd = c_per_w - 4 + s
            cd = base + gd
            fd = lax.div(cd, n_bblk)
            bd = lax.rem(cd, n_bblk)
            pltpu.make_async_copy(ebuf.at[gd & 3], emb_out.at[fd, bd],
                                  sem_eo.at[gd & 3]).wait()
            pltpu.make_async_copy(lvbuf.at[gd & 3], lin_out.at[fd, bd],
                                  sem_lo.at[gd & 3]).wait()

    return gather_kernel(idx3d, q128, l128)


def _tc_head(emb4, lin3, s_t, w1t, b1c, w2t, b2c, w3c, cbias):
    f, n_bblk = emb4.shape[0], emb4.shape[1]
    d_in = f * 16
    h1 = w1t.shape[0]
    h2 = w2t.shape[0]

    nb = 8

    def body(emb_ref, lin_ref, s_ref, w1_ref, b1_ref, w2_ref, b2_ref,
             w3_ref, cb_ref, out_ref):
        x = jnp.concatenate(
            [emb_ref[:, q].reshape(d_in, 128) for q in range(nb)],
            axis=1)
        ksum = jnp.dot(s_ref[...], x, preferred_element_type=jnp.float32)
        sq_sum = jnp.sum(ksum * ksum, axis=0, keepdims=True)
        sum_sq = jnp.sum(x * x, axis=0, keepdims=True)
        quad = 0.5 * (sq_sum - sum_sq)
        lin = jnp.concatenate(
            [jnp.sum(lin_ref[:, q, 0, :], axis=0, keepdims=True)
             for q in range(nb)], axis=1)
        h = jnp.dot(w1_ref[...], x, preferred_element_type=jnp.float32)
        h = jnp.maximum(h + b1_ref[...], 0.0)
        h = jnp.dot(w2_ref[...], h, preferred_element_type=jnp.float32)
        h = jnp.maximum(h + b2_ref[...], 0.0)
        ymlp = jnp.sum(h * w3_ref[...], axis=0, keepdims=True)
        z = cb_ref[...] + lin + quad + ymlp
        out_ref[...] = (1.0 / (1.0 + jnp.exp(-z))).reshape(1, 1, nb * 128)

    return pl.pallas_call(
        body,
        grid=(n_bblk // 8,),
        in_specs=[
            pl.BlockSpec((f, 8, 16, 128), lambda i: (0, i, 0, 0)),
            pl.BlockSpec((f, 8, 1, 128), lambda i: (0, i, 0, 0)),
            pl.BlockSpec((16, d_in), lambda i: (0, 0)),
            pl.BlockSpec((h1, d_in), lambda i: (0, 0)),
            pl.BlockSpec((h1, 1), lambda i: (0, 0)),
            pl.BlockSpec((h2, h1), lambda i: (0, 0)),
            pl.BlockSpec((h2, 1), lambda i: (0, 0)),
            pl.BlockSpec((h2, 1), lambda i: (0, 0)),
            pl.BlockSpec((1, 1), lambda i: (0, 0)),
        ],
        out_specs=pl.BlockSpec((1, 1, 1024), lambda i: (i, 0, 0)),
        out_shape=jax.ShapeDtypeStruct((n_bblk // 8, 1, 1024), jnp.float32),
    )(emb4, lin3, s_t, w1t, b1c, w2t, b2c, w3c, cbias)


def kernel(input, quad_table, lin_table, global_bias, W1, b1, W2, b2, W3, b3):
    b, f = input.shape
    r, k = quad_table.shape
    vocab = r // f
    nw = 32
    n_bblk = b // 128
    offsets = jnp.arange(f, dtype=input.dtype) * vocab
    idx_fm = input.T + offsets[:, None]
    idx3d = idx_fm.reshape(nw, -1, 128)
    q128, l128 = _tc_pack(quad_table.T, lin_table.T)
    emb4, lin3 = _sc_gather(idx3d, q128.reshape(-1, 16),
                            l128.reshape(-1, 16), f, n_bblk)
    s_t = jnp.tile(jnp.eye(k, dtype=jnp.float32), (1, f))
    cbias = (global_bias[0] + b3[0]).reshape(1, 1)
    out = _tc_head(emb4, lin3, s_t, W1.T, b1.reshape(-1, 1), W2.T,
                   b2.reshape(-1, 1), W3, cbias)
    return out.reshape(b)
